# baseline (speedup 1.0000x reference)
import jax
import jax.numpy as jnp
from jax import lax
from jax.experimental import pallas as pl
from jax.experimental.pallas import tpu as pltpu
from jax.experimental.pallas import tpu_sc as plsc

VARIANCE_FOCUS = 0.85
LN2 = 0.6931471805599453
SQRT2 = 1.4142135623730951

_N = 16 * 512 * 512
_COLS = 512
_ROWS = _N // _COLS

_NW = 32
_SC_ROWS_PER_W = 32
_SC_ROWS = _NW * _SC_ROWS_PER_W
_TC_ROWS = _ROWS - _SC_ROWS
_TC_GRID = 4
_TC_BLK = _TC_ROWS // _TC_GRID

_VPW = _SC_ROWS_PER_W * _COLS // 16


def _tc_body(est_ref, gt_ref, part_ref, acc_ref):
    i = pl.program_id(0)

    @pl.when(i == 0)
    def _init():
        acc_ref[0] = 0.0
        acc_ref[1] = 0.0
        acc_ref[2] = 0.0

    est = est_ref[...]
    gt = gt_ref[...]
    mask = gt > 1.0
    d = jnp.where(
        mask,
        jnp.log(jnp.where(mask, est, 1.0)) - jnp.log(jnp.where(mask, gt, 1.0)),
        0.0,
    )
    acc_ref[0] += jnp.sum(d)
    acc_ref[1] += jnp.sum(d * d)
    acc_ref[2] += jnp.sum(mask.astype(jnp.float32))

    @pl.when(i == _TC_GRID - 1)
    def _fin():
        part_ref[0] = acc_ref[0]
        part_ref[1] = acc_ref[1]
        part_ref[2] = acc_ref[2]


def _fast_log(x):
    bits = lax.bitcast_convert_type(x, jnp.int32)
    e = lax.shift_right_logical(bits, 23) - 127
    mbits = lax.bitwise_or(
        lax.bitwise_and(bits, jnp.int32(0x7FFFFF)), jnp.int32(0x3F800000)
    )
    m = lax.bitcast_convert_type(mbits, jnp.float32)
    big = m > SQRT2
    m = jnp.where(big, m * 0.5, m)
    e = jnp.where(big, e + 1, e)
    ef = e.astype(jnp.float32)
    s = (m - 1.0) / (m + 1.0)
    z = s * s
    p = jnp.float32(1.0 / 9.0)
    p = p * z + jnp.float32(1.0 / 7.0)
    p = p * z + jnp.float32(1.0 / 5.0)
    p = p * z + jnp.float32(1.0 / 3.0)
    p = p * z + 1.0
    return ef * LN2 + 2.0 * s * p


def _sc_body(est_hbm, gt_hbm, part_hbm, est_v, gt_v, out_v, sem):
    wid = lax.axis_index("c") * 16 + lax.axis_index("s")
    base_row = _TC_ROWS + wid * _SC_ROWS_PER_W
    pltpu.sync_copy(est_hbm.at[pl.ds(base_row, _SC_ROWS_PER_W), :], est_v)
    pltpu.sync_copy(gt_hbm.at[pl.ds(base_row, _SC_ROWS_PER_W), :], gt_v)

    def vec_step(i, accs):
        acc_d, acc_d2, acc_n = accs
        r = lax.shift_right_logical(i, 5)
        c = pl.multiple_of(lax.shift_left(lax.bitwise_and(i, 31), 4), 16)
        est = est_v[r, pl.ds(c, 16)]
        gt = gt_v[r, pl.ds(c, 16)]
        mask = gt > 1.0
        ratio = jnp.where(mask, est / gt, 1.0)
        d = _fast_log(ratio)
        acc_d = acc_d + d
        acc_d2 = acc_d2 + d * d
        acc_n = acc_n + jnp.where(mask, 1.0, 0.0)
        return (acc_d, acc_d2, acc_n)

    zero = jnp.zeros((16,), jnp.float32)
    acc_d, acc_d2, acc_n = lax.fori_loop(0, _VPW, vec_step, (zero, zero, zero))
    out_v[pl.ds(0, 16)] = acc_d
    out_v[pl.ds(16, 16)] = acc_d2
    out_v[pl.ds(32, 16)] = acc_n
    pltpu.sync_copy(out_v, part_hbm.at[wid])


def _combine_body(tc_ref, sc_ref, out_ref):
    p = sc_ref[...]
    sd = tc_ref[0] + jnp.sum(p[:, 0:16])
    sd2 = tc_ref[1] + jnp.sum(p[:, 16:32])
    n = tc_ref[2] + jnp.sum(p[:, 32:48])
    mean_d = sd / n
    mean_d2 = sd2 / n
    out_ref[0] = jnp.sqrt(mean_d2 - VARIANCE_FOCUS * mean_d * mean_d) * 10.0


def kernel(depth_est, depth_gt):
    est2d = depth_est.reshape(_ROWS, _COLS)
    gt2d = depth_gt.reshape(_ROWS, _COLS)

    mesh = plsc.VectorSubcoreMesh(core_axis_name="c", subcore_axis_name="s")
    sc_part = pl.kernel(
        _sc_body,
        mesh=mesh,
        out_type=jax.ShapeDtypeStruct((_NW, 48), jnp.float32),
        scratch_types=[
            pltpu.VMEM((_SC_ROWS_PER_W, _COLS), jnp.float32),
            pltpu.VMEM((_SC_ROWS_PER_W, _COLS), jnp.float32),
            pltpu.VMEM((48,), jnp.float32),
            pltpu.SemaphoreType.DMA,
        ],
        compiler_params=pltpu.CompilerParams(use_tc_tiling_on_sc=True),
        cost_estimate=pl.CostEstimate(
            flops=20 * _SC_ROWS * _COLS,
            bytes_accessed=2 * 4 * _SC_ROWS * _COLS,
            transcendentals=0,
        ),
    )(est2d, gt2d)

    tc_part = pl.pallas_call(
        _tc_body,
        grid=(_TC_GRID,),
        in_specs=[
            pl.BlockSpec((_TC_BLK, _COLS), lambda i: (i, 0)),
            pl.BlockSpec((_TC_BLK, _COLS), lambda i: (i, 0)),
        ],
        out_specs=pl.BlockSpec(memory_space=pltpu.SMEM),
        out_shape=jax.ShapeDtypeStruct((3,), jnp.float32),
        scratch_shapes=[pltpu.SMEM((3,), jnp.float32)],
    )(est2d, gt2d)

    out = pl.pallas_call(
        _combine_body,
        in_specs=[
            pl.BlockSpec(memory_space=pltpu.SMEM),
            pl.BlockSpec((_NW, 48), lambda: (0, 0)),
        ],
        out_specs=pl.BlockSpec(memory_space=pltpu.SMEM),
        out_shape=jax.ShapeDtypeStruct((1,), jnp.float32),
    )(tc_part, sc_part)
    return out[0]


# speedup vs baseline: 1.0233x; 1.0233x over previous
import jax
import jax.numpy as jnp
from jax import lax
from jax.experimental import pallas as pl
from jax.experimental.pallas import tpu as pltpu
from jax.experimental.pallas import tpu_sc as plsc

VARIANCE_FOCUS = 0.85
LN2 = 0.6931471805599453
SQRT2 = 1.4142135623730951

_N = 16 * 512 * 512
_COLS = 512
_ROWS = _N // _COLS

_NW = 32
_SC_ROWS_PER_W = 32
_SC_ROWS = _NW * _SC_ROWS_PER_W
_TC_ROWS = _ROWS - _SC_ROWS
_TC_GRID = 4
_TC_BLK = _TC_ROWS // _TC_GRID

_VPW = _SC_ROWS_PER_W * _COLS // 16


def _tc_body(est_ref, gt_ref, part_ref, acc_ref):
    i = pl.program_id(0)

    @pl.when(i == 0)
    def _init():
        acc_ref[0] = 0.0
        acc_ref[1] = 0.0
        acc_ref[2] = 0.0

    est = est_ref[...]
    gt = gt_ref[...]
    mask = gt > 1.0
    d = jnp.where(
        mask,
        jnp.log(jnp.where(mask, est, 1.0)) - jnp.log(jnp.where(mask, gt, 1.0)),
        0.0,
    )
    acc_ref[0] += jnp.sum(d)
    acc_ref[1] += jnp.sum(d * d)
    acc_ref[2] += jnp.sum(mask.astype(jnp.float32))

    @pl.when(i == _TC_GRID - 1)
    def _fin():
        part_ref[0] = acc_ref[0]
        part_ref[1] = acc_ref[1]
        part_ref[2] = acc_ref[2]


def _fast_log(x):
    bits = lax.bitcast_convert_type(x, jnp.int32)
    e = lax.shift_right_logical(bits, 23) - 127
    mbits = lax.bitwise_or(
        lax.bitwise_and(bits, jnp.int32(0x7FFFFF)), jnp.int32(0x3F800000)
    )
    m = lax.bitcast_convert_type(mbits, jnp.float32)
    big = m > SQRT2
    m = jnp.where(big, m * 0.5, m)
    e = jnp.where(big, e + 1, e)
    ef = e.astype(jnp.float32)
    s = (m - 1.0) / (m + 1.0)
    z = s * s
    p = jnp.float32(1.0 / 9.0)
    p = p * z + jnp.float32(1.0 / 7.0)
    p = p * z + jnp.float32(1.0 / 5.0)
    p = p * z + jnp.float32(1.0 / 3.0)
    p = p * z + 1.0
    return ef * LN2 + 2.0 * s * p


def _sc_body(est_hbm, gt_hbm, part_hbm, est_v, gt_v, out_v, sem):
    wid = lax.axis_index("c") * 16 + lax.axis_index("s")
    base_row = _TC_ROWS + wid * _SC_ROWS_PER_W
    pltpu.sync_copy(est_hbm.at[pl.ds(base_row, _SC_ROWS_PER_W), :], est_v)
    pltpu.sync_copy(gt_hbm.at[pl.ds(base_row, _SC_ROWS_PER_W), :], gt_v)

    def vec_step(i, accs):
        acc_d, acc_d2, acc_n = accs
        r = lax.shift_right_logical(i, 5)
        c = pl.multiple_of(lax.shift_left(lax.bitwise_and(i, 31), 4), 16)
        est = est_v[r, pl.ds(c, 16)]
        gt = gt_v[r, pl.ds(c, 16)]
        mask = gt > 1.0
        ratio = jnp.where(mask, est / gt, 1.0)
        d = _fast_log(ratio)
        acc_d = acc_d + d
        acc_d2 = acc_d2 + d * d
        acc_n = acc_n + jnp.where(mask, 1.0, 0.0)
        return (acc_d, acc_d2, acc_n)

    zero = jnp.zeros((16,), jnp.float32)
    acc_d, acc_d2, acc_n = lax.fori_loop(0, _VPW, vec_step, (zero, zero, zero))
    out_v[pl.ds(0, 16)] = acc_d
    out_v[pl.ds(16, 16)] = acc_d2
    out_v[pl.ds(32, 16)] = acc_n
    pltpu.sync_copy(out_v, part_hbm.at[wid])


def _combine_body(tc_ref, sc_ref, out_ref):
    p = sc_ref[...]
    sd = tc_ref[0] + jnp.sum(p[:, 0:16])
    sd2 = tc_ref[1] + jnp.sum(p[:, 16:32])
    n = tc_ref[2] + jnp.sum(p[:, 32:48])
    mean_d = sd / n
    mean_d2 = sd2 / n
    out_ref[0] = jnp.sqrt(mean_d2 - VARIANCE_FOCUS * mean_d * mean_d) * 10.0


def kernel(depth_est, depth_gt):
    est2d = depth_est.reshape(_ROWS, _COLS)
    gt2d = depth_gt.reshape(_ROWS, _COLS)

    mesh = plsc.VectorSubcoreMesh(core_axis_name="c", subcore_axis_name="s")
    sc_part = pl.kernel(
        _sc_body,
        mesh=mesh,
        out_type=jax.ShapeDtypeStruct((_NW, 48), jnp.float32),
        scratch_types=[
            pltpu.VMEM((_SC_ROWS_PER_W, _COLS), jnp.float32),
            pltpu.VMEM((_SC_ROWS_PER_W, _COLS), jnp.float32),
            pltpu.VMEM((48,), jnp.float32),
            pltpu.SemaphoreType.DMA,
        ],
        compiler_params=pltpu.CompilerParams(
            use_tc_tiling_on_sc=True, skip_device_barrier=True
        ),
        cost_estimate=pl.CostEstimate(
            flops=20 * _SC_ROWS * _COLS,
            bytes_accessed=2 * 4 * _SC_ROWS * _COLS,
            transcendentals=0,
        ),
    )(est2d, gt2d)

    tc_part = pl.pallas_call(
        _tc_body,
        grid=(_TC_GRID,),
        in_specs=[
            pl.BlockSpec((_TC_BLK, _COLS), lambda i: (i, 0)),
            pl.BlockSpec((_TC_BLK, _COLS), lambda i: (i, 0)),
        ],
        out_specs=pl.BlockSpec(memory_space=pltpu.SMEM),
        out_shape=jax.ShapeDtypeStruct((3,), jnp.float32),
        scratch_shapes=[pltpu.SMEM((3,), jnp.float32)],
    )(est2d, gt2d)

    out = pl.pallas_call(
        _combine_body,
        in_specs=[
            pl.BlockSpec(memory_space=pltpu.SMEM),
            pl.BlockSpec((_NW, 48), lambda: (0, 0)),
        ],
        out_specs=pl.BlockSpec(memory_space=pltpu.SMEM),
        out_shape=jax.ShapeDtypeStruct((1,), jnp.float32),
    )(tc_part, sc_part)
    return out[0]


# speedup vs baseline: 1.9222x; 1.8785x over previous
import jax
import jax.numpy as jnp
from jax.experimental import pallas as pl
from jax.experimental.pallas import tpu as pltpu

VARIANCE_FOCUS = 0.85

_ROWS = 8192
_COLS = 512
_BLK_ROWS = 2048
_GRID = _ROWS // _BLK_ROWS


def _silog_body(est_ref, gt_ref, out_ref, acc_ref):
    i = pl.program_id(0)

    @pl.when(i == 0)
    def _init():
        acc_ref[0] = 0.0
        acc_ref[1] = 0.0
        acc_ref[2] = 0.0

    est = est_ref[...]
    gt = gt_ref[...]
    mask = gt > 1.0
    d = jnp.where(
        mask,
        jnp.log(jnp.where(mask, est, 1.0)) - jnp.log(jnp.where(mask, gt, 1.0)),
        0.0,
    )
    acc_ref[0] += jnp.sum(d)
    acc_ref[1] += jnp.sum(d * d)
    acc_ref[2] += jnp.sum(mask.astype(jnp.float32))

    @pl.when(i == _GRID - 1)
    def _fin():
        n = acc_ref[2]
        mean_d = acc_ref[0] / n
        mean_d2 = acc_ref[1] / n
        out_ref[0] = jnp.sqrt(mean_d2 - VARIANCE_FOCUS * mean_d * mean_d) * 10.0


def kernel(depth_est, depth_gt):
    est2d = depth_est.reshape(_ROWS, _COLS)
    gt2d = depth_gt.reshape(_ROWS, _COLS)
    out = pl.pallas_call(
        _silog_body,
        grid=(_GRID,),
        in_specs=[
            pl.BlockSpec((_BLK_ROWS, _COLS), lambda i: (i, 0)),
            pl.BlockSpec((_BLK_ROWS, _COLS), lambda i: (i, 0)),
        ],
        out_specs=pl.BlockSpec(memory_space=pltpu.SMEM),
        out_shape=jax.ShapeDtypeStruct((1,), jnp.float32),
        scratch_shapes=[pltpu.SMEM((3,), jnp.float32)],
    )(est2d, gt2d)
    return out[0]
